# baseline (speedup 1.0000x reference)
import jax
import jax.numpy as jnp
from jax.experimental import pallas as pl
from jax.experimental.pallas import tpu as pltpu

NUM_CLASSES = 1000
D = 256
B = 4096
KPAD = 1024
BB = 512
NBLK = B // BB


def _tc_body(x_ref, lab_ref, cent_ref, out_ref, cbf_ref):
    i = pl.program_id(0)

    @pl.when(i == 0)
    def _prep():
        cb = cent_ref[...].astype(jnp.bfloat16)
        pad = jnp.zeros((KPAD - NUM_CLASSES, D), jnp.bfloat16)
        cbf_ref[...] = jnp.concatenate([cb, pad], axis=0)

    labs = lab_ref[0, 0, :]
    iota_k = jax.lax.broadcasted_iota(jnp.int32, (BB, KPAD), 1)
    onehot = (labs[:, None] == iota_k).astype(jnp.bfloat16)
    g = jnp.dot(onehot, cbf_ref[...],
                preferred_element_type=jnp.float32)
    d = x_ref[...] - g
    part = jnp.sum(d * d).reshape(1, 1)

    @pl.when(i == 0)
    def _init():
        out_ref[...] = part

    @pl.when(i != 0)
    def _acc():
        out_ref[...] += part

    @pl.when(i == NBLK - 1)
    def _fin():
        out_ref[...] = out_ref[...] * (1.0 / B)


def kernel(x, labels, centers):
    labels_i32 = labels.astype(jnp.int32)
    loss = pl.pallas_call(
        _tc_body,
        grid=(NBLK,),
        in_specs=[
            pl.BlockSpec((BB, D), lambda i: (i, 0)),
            pl.BlockSpec((1, 1, BB), lambda i: (i, 0, 0)),
            pl.BlockSpec((NUM_CLASSES, D), lambda i: (0, 0)),
        ],
        out_specs=pl.BlockSpec((1, 1), lambda i: (0, 0)),
        out_shape=jax.ShapeDtypeStruct((1, 1), jnp.float32),
        scratch_shapes=[pltpu.VMEM((KPAD, D), jnp.bfloat16)],
    )(x, labels_i32.reshape(NBLK, 1, BB), centers)
    return loss[0, 0]


# speedup vs baseline: 1.5323x; 1.5323x over previous
import jax
import jax.numpy as jnp
from jax.experimental import pallas as pl
from jax.experimental.pallas import tpu as pltpu

NUM_CLASSES = 1000
D = 256
B = 4096
KPAD = 1024
BB = 2048
NBLK = B // BB


def _tc_body(x_ref, lab_ref, cent_ref, out_ref, cbf_ref):
    i = pl.program_id(0)

    @pl.when(i == 0)
    def _prep():
        cb = cent_ref[...].astype(jnp.bfloat16)
        pad = jnp.zeros((KPAD - NUM_CLASSES, D), jnp.bfloat16)
        cbf_ref[...] = jnp.concatenate([cb, pad], axis=0)

    labs = lab_ref[0, 0, :]
    iota_k = jax.lax.broadcasted_iota(jnp.int32, (BB, KPAD), 1)
    onehot = (labs[:, None] == iota_k).astype(jnp.bfloat16)
    g = jnp.dot(onehot, cbf_ref[...],
                preferred_element_type=jnp.float32)
    d = x_ref[...] - g
    part = jnp.sum(d * d).reshape(1, 1)

    @pl.when(i == 0)
    def _init():
        out_ref[...] = part

    @pl.when(i != 0)
    def _acc():
        out_ref[...] += part

    @pl.when(i == NBLK - 1)
    def _fin():
        out_ref[...] = out_ref[...] * (1.0 / B)


def kernel(x, labels, centers):
    labels_i32 = labels.astype(jnp.int32)
    loss = pl.pallas_call(
        _tc_body,
        grid=(NBLK,),
        in_specs=[
            pl.BlockSpec((BB, D), lambda i: (i, 0)),
            pl.BlockSpec((1, 1, BB), lambda i: (i, 0, 0)),
            pl.BlockSpec((NUM_CLASSES, D), lambda i: (0, 0)),
        ],
        out_specs=pl.BlockSpec((1, 1), lambda i: (0, 0)),
        out_shape=jax.ShapeDtypeStruct((1, 1), jnp.float32),
        scratch_shapes=[pltpu.VMEM((KPAD, D), jnp.bfloat16)],
    )(x, labels_i32.reshape(NBLK, 1, BB), centers)
    return loss[0, 0]
